# baseline (speedup 1.0000x reference)
import functools

import jax
import jax.numpy as jnp
from jax import lax
from jax.experimental import pallas as pl
from jax.experimental.pallas import tpu as pltpu
from jax.experimental.pallas import tpu_sc as plsc

N_ELECTRODES = 100000
EMBED_DIM = 128

NUM_WORKERS = 32
CHUNK = 128
CHUNKS_PER_W = 25
ROWS_PER_W = CHUNK * CHUNKS_PER_W
B_PAD = NUM_WORKERS * ROWS_PER_W

_mesh = plsc.VectorSubcoreMesh(core_axis_name="c", subcore_axis_name="s")


@functools.partial(
    pl.kernel,
    out_type=jax.ShapeDtypeStruct((B_PAD, EMBED_DIM), jnp.float32),
    mesh=_mesh,
    scratch_types=[
        pltpu.VMEM((CHUNKS_PER_W, CHUNK), jnp.int32),
        pltpu.VMEM((5, CHUNK, EMBED_DIM), jnp.float32),
        pltpu.SemaphoreType.DMA,
        pltpu.SemaphoreType.DMA,
        pltpu.SemaphoreType.DMA,
        pltpu.SemaphoreType.DMA,
        pltpu.SemaphoreType.DMA,
        pltpu.SemaphoreType.DMA,
        pltpu.SemaphoreType.DMA,
        pltpu.SemaphoreType.DMA,
        pltpu.SemaphoreType.DMA,
        pltpu.SemaphoreType.DMA,
    ],
)
def _gather_kernel(table_hbm, idx_hbm, out_hbm, idx_v, bufs,
                   g0, g1, g2, g3, g4, w0, w1, w2, w3, w4):
    wid = lax.axis_index("s") * 2 + lax.axis_index("c")
    base = wid * ROWS_PER_W

    pltpu.sync_copy(idx_hbm.at[wid], idx_v)

    NBUF = 5
    gsems = (g0, g1, g2, g3, g4)
    wsems = (w0, w1, w2, w3, w4)

    def gather_start(j, b):
        pltpu.async_copy(table_hbm.at[idx_v.at[j]], bufs.at[b], gsems[b])

    def gather_wait(b):
        pltpu.make_async_copy(table_hbm.at[idx_v.at[0]], bufs.at[b],
                              gsems[b]).wait()

    def write_start(j, b):
        pltpu.async_copy(bufs.at[b],
                         out_hbm.at[pl.ds(base + j * CHUNK, CHUNK)], wsems[b])

    def write_wait(b):
        pltpu.make_async_copy(bufs.at[b], out_hbm.at[pl.ds(base, CHUNK)],
                              wsems[b]).wait()

    for j in range(NBUF - 1):
        gather_start(j, j)

    def chunk_body(jj, k):
        b = k % NBUF
        bg = (k + NBUF - 1) % NBUF

        @pl.when(jj >= 1)
        def _():
            write_wait(bg)

        @pl.when(jj + NBUF - 1 < CHUNKS_PER_W)
        def _():
            gather_start(jj + NBUF - 1, bg)

        gather_wait(b)
        write_start(jj, b)

    def outer(i, carry):
        jj0 = i * NBUF
        for k in range(NBUF):
            chunk_body(jj0 + k, k)
        return carry

    lax.fori_loop(0, CHUNKS_PER_W // NBUF, outer, 0)

    write_wait((CHUNKS_PER_W - 1) % NBUF)


def kernel(table, permutation, subject_id=0):
    idx = permutation.astype(jnp.int32)
    idx_pad = jnp.pad(idx, (0, B_PAD - idx.shape[0]))
    idx3d = idx_pad.reshape(NUM_WORKERS, CHUNKS_PER_W, CHUNK)
    out = _gather_kernel(table, idx3d)
    return out[:N_ELECTRODES]


# speedup vs baseline: 2.5718x; 2.5718x over previous
import functools

import jax
import jax.numpy as jnp
from jax import lax
from jax.experimental import pallas as pl
from jax.experimental.pallas import tpu as pltpu
from jax.experimental.pallas import tpu_sc as plsc

N_ELECTRODES = 100000
EMBED_DIM = 128

NUM_WORKERS = 32
CHUNK = 128
CHUNKS_PER_W = 25
ROWS_PER_W = CHUNK * CHUNKS_PER_W
B_PAD = NUM_WORKERS * ROWS_PER_W

_mesh = plsc.VectorSubcoreMesh(core_axis_name="c", subcore_axis_name="s")


@functools.partial(
    pl.kernel,
    out_type=jax.ShapeDtypeStruct((B_PAD, EMBED_DIM), jnp.float32),
    mesh=_mesh,
    scratch_types=[
        pltpu.VMEM((CHUNKS_PER_W, CHUNK), jnp.int32),
        pltpu.VMEM((5, CHUNK, EMBED_DIM), jnp.float32),
        pltpu.SemaphoreType.DMA,
        pltpu.SemaphoreType.DMA,
        pltpu.SemaphoreType.DMA,
        pltpu.SemaphoreType.DMA,
        pltpu.SemaphoreType.DMA,
        pltpu.SemaphoreType.DMA,
        pltpu.SemaphoreType.DMA,
        pltpu.SemaphoreType.DMA,
        pltpu.SemaphoreType.DMA,
        pltpu.SemaphoreType.DMA,
    ],
)
def _gather_kernel(table_hbm, idx_hbm, out_hbm, idx_v, bufs,
                   g0, g1, g2, g3, g4, w0, w1, w2, w3, w4):
    wid = lax.axis_index("s") * 2 + lax.axis_index("c")
    base = wid * ROWS_PER_W

    pltpu.sync_copy(idx_hbm.at[wid], idx_v)

    NBUF = 5
    gsems = (g0, g1, g2, g3, g4)
    wsems = (w0, w1, w2, w3, w4)

    def gather_start(j, b):
        pltpu.async_copy(table_hbm.at[idx_v.at[j]], bufs.at[b], gsems[b])

    def gather_wait(b):
        pltpu.make_async_copy(table_hbm.at[idx_v.at[0]], bufs.at[b],
                              gsems[b]).wait()

    def write_start(j, b):
        pltpu.async_copy(bufs.at[b],
                         out_hbm.at[pl.ds(base + j * CHUNK, CHUNK)], wsems[b])

    def write_wait(b):
        pltpu.make_async_copy(bufs.at[b], out_hbm.at[pl.ds(base, CHUNK)],
                              wsems[b]).wait()

    for j in range(NBUF - 1):
        gather_start(j, j)

    def chunk_body(jj, k):
        b = k % NBUF
        bg = (k + NBUF - 1) % NBUF

        @pl.when(jj >= 1)
        def _():
            write_wait(bg)

        @pl.when(jj + NBUF - 1 < CHUNKS_PER_W)
        def _():
            gather_start(jj + NBUF - 1, bg)

        gather_wait(b)
        write_start(jj, b)

    def outer(i, carry):
        jj0 = i * NBUF
        for k in range(NBUF):
            chunk_body(jj0 + k, k)
        return carry

    lax.fori_loop(0, CHUNKS_PER_W // NBUF, outer, 0)

    write_wait((CHUNKS_PER_W - 1) % NBUF)


def kernel(table, permutation, subject_id=0):
    idx = permutation.astype(jnp.int32)
    n_pad = B_PAD - idx.shape[0]
    idx_pad = jnp.concatenate([idx, jnp.arange(n_pad, dtype=jnp.int32)])
    idx3d = idx_pad.reshape(NUM_WORKERS, CHUNKS_PER_W, CHUNK)
    out = _gather_kernel(table, idx3d)
    return out[:N_ELECTRODES]


# speedup vs baseline: 4.0521x; 1.5756x over previous
import functools

import jax
import jax.numpy as jnp
from jax import lax
from jax.experimental import pallas as pl
from jax.experimental.pallas import tpu as pltpu
from jax.experimental.pallas import tpu_sc as plsc

N_ELECTRODES = 100000
EMBED_DIM = 128

NUM_WORKERS = 32
CHUNK = 128
FULL_CHUNKS = 24
BIG_ROWS = 3128
SMALL_ROWS = 3120
NUM_BIG = 20
BIG_TAIL = BIG_ROWS - FULL_CHUNKS * CHUNK
SMALL_TAIL = SMALL_ROWS - FULL_CHUNKS * CHUNK
IDX_PAD = 8
NBUF = 4

_mesh = plsc.VectorSubcoreMesh(core_axis_name="c", subcore_axis_name="s")


@functools.partial(
    pl.kernel,
    out_type=jax.ShapeDtypeStruct((N_ELECTRODES, EMBED_DIM), jnp.float32),
    mesh=_mesh,
    scratch_types=[
        pltpu.VMEM((BIG_ROWS,), jnp.int32),
        pltpu.VMEM((NBUF, CHUNK, EMBED_DIM), jnp.float32),
        pltpu.VMEM((BIG_TAIL, EMBED_DIM), jnp.float32),
        pltpu.SemaphoreType.DMA,
        pltpu.SemaphoreType.DMA,
        pltpu.SemaphoreType.DMA,
        pltpu.SemaphoreType.DMA,
        pltpu.SemaphoreType.DMA,
        pltpu.SemaphoreType.DMA,
        pltpu.SemaphoreType.DMA,
        pltpu.SemaphoreType.DMA,
        pltpu.SemaphoreType.DMA,
        pltpu.SemaphoreType.DMA,
    ],
)
def _gather_kernel(table_hbm, idx_hbm, out_hbm, idx_v, bufs, tail_buf,
                   g0, g1, g2, g3, w0, w1, w2, w3, gt, wt):
    wid = lax.axis_index("s") * 2 + lax.axis_index("c")
    is_big = wid < NUM_BIG
    base = jnp.where(is_big, wid * BIG_ROWS,
                     NUM_BIG * BIG_ROWS + (wid - NUM_BIG) * SMALL_ROWS)

    pltpu.sync_copy(idx_hbm.at[pl.ds(base, BIG_ROWS)], idx_v)

    gsems = (g0, g1, g2, g3)
    wsems = (w0, w1, w2, w3)

    def gather_start(j, b):
        pltpu.async_copy(table_hbm.at[idx_v.at[pl.ds(j * CHUNK, CHUNK)]],
                         bufs.at[b], gsems[b])

    def gather_wait(b):
        pltpu.make_async_copy(table_hbm.at[idx_v.at[pl.ds(0, CHUNK)]],
                              bufs.at[b], gsems[b]).wait()

    def write_start(j, b):
        pltpu.async_copy(bufs.at[b],
                         out_hbm.at[pl.ds(base + j * CHUNK, CHUNK)], wsems[b])

    def write_wait(b):
        pltpu.make_async_copy(bufs.at[b], out_hbm.at[pl.ds(base, CHUNK)],
                              wsems[b]).wait()

    pltpu.async_copy(table_hbm.at[idx_v.at[pl.ds(FULL_CHUNKS * CHUNK,
                                                 BIG_TAIL)]], tail_buf, gt)

    for j in range(NBUF - 1):
        gather_start(j, j)

    def chunk_body(jj, k):
        b = k % NBUF
        bg = (k + NBUF - 1) % NBUF

        @pl.when(jj >= 1)
        def _():
            write_wait(bg)

        @pl.when(jj + NBUF - 1 < FULL_CHUNKS)
        def _():
            gather_start(jj + NBUF - 1, bg)

        gather_wait(b)
        write_start(jj, b)

    def outer(i, carry):
        jj0 = i * NBUF
        for k in range(NBUF):
            chunk_body(jj0 + k, k)
        return carry

    lax.fori_loop(0, FULL_CHUNKS // NBUF, outer, 0)

    write_wait((FULL_CHUNKS - 1) % NBUF)

    pltpu.make_async_copy(table_hbm.at[idx_v.at[pl.ds(0, BIG_TAIL)]],
                          tail_buf, gt).wait()
    tail_off = base + FULL_CHUNKS * CHUNK

    @pl.when(is_big)
    def _():
        pltpu.async_copy(tail_buf, out_hbm.at[pl.ds(tail_off, BIG_TAIL)], wt)
        pltpu.make_async_copy(tail_buf, out_hbm.at[pl.ds(0, BIG_TAIL)],
                              wt).wait()

    @pl.when(jnp.logical_not(is_big))
    def _():
        pltpu.async_copy(tail_buf.at[pl.ds(0, SMALL_TAIL)],
                         out_hbm.at[pl.ds(tail_off, SMALL_TAIL)], wt)
        pltpu.make_async_copy(tail_buf.at[pl.ds(0, SMALL_TAIL)],
                              out_hbm.at[pl.ds(0, SMALL_TAIL)], wt).wait()


def kernel(table, permutation, subject_id=0):
    idx = permutation.astype(jnp.int32)
    idx_pad = jnp.concatenate(
        [idx, jnp.arange(IDX_PAD, dtype=jnp.int32)])
    return _gather_kernel(table, idx_pad)


# speedup vs baseline: 4.1084x; 1.0139x over previous
import functools

import jax
import jax.numpy as jnp
from jax import lax
from jax.experimental import pallas as pl
from jax.experimental.pallas import tpu as pltpu
from jax.experimental.pallas import tpu_sc as plsc

N_ELECTRODES = 100000
EMBED_DIM = 128

NUM_WORKERS = 32
CHUNK = 128
FULL_CHUNKS = 24
SMALL_ROWS = 3120
BIG_ROWS = 3128
NUM_SMALL = 12
BIG_TAIL = BIG_ROWS - FULL_CHUNKS * CHUNK
SMALL_TAIL = SMALL_ROWS - FULL_CHUNKS * CHUNK
NBUF = 4

_mesh = plsc.VectorSubcoreMesh(core_axis_name="c", subcore_axis_name="s")


@functools.partial(
    pl.kernel,
    out_type=jax.ShapeDtypeStruct((N_ELECTRODES, EMBED_DIM), jnp.float32),
    mesh=_mesh,
    scratch_types=[
        pltpu.VMEM((BIG_ROWS,), jnp.int32),
        pltpu.VMEM((NBUF, CHUNK, EMBED_DIM), jnp.float32),
        pltpu.VMEM((BIG_TAIL, EMBED_DIM), jnp.float32),
        pltpu.SemaphoreType.DMA,
        pltpu.SemaphoreType.DMA,
        pltpu.SemaphoreType.DMA,
        pltpu.SemaphoreType.DMA,
        pltpu.SemaphoreType.DMA,
        pltpu.SemaphoreType.DMA,
        pltpu.SemaphoreType.DMA,
        pltpu.SemaphoreType.DMA,
        pltpu.SemaphoreType.DMA,
        pltpu.SemaphoreType.DMA,
    ],
)
def _gather_kernel(table_hbm, idx_hbm, out_hbm, idx_v, bufs, tail_buf,
                   g0, g1, g2, g3, w0, w1, w2, w3, gt, wt):
    wid = lax.axis_index("s") * 2 + lax.axis_index("c")
    is_big = wid >= NUM_SMALL
    base = jnp.where(is_big,
                     NUM_SMALL * SMALL_ROWS + (wid - NUM_SMALL) * BIG_ROWS,
                     wid * SMALL_ROWS)

    pltpu.sync_copy(idx_hbm.at[pl.ds(base, BIG_ROWS)], idx_v)

    gsems = (g0, g1, g2, g3)
    wsems = (w0, w1, w2, w3)

    def gather_start(j, b):
        pltpu.async_copy(table_hbm.at[idx_v.at[pl.ds(j * CHUNK, CHUNK)]],
                         bufs.at[b], gsems[b])

    def gather_wait(b):
        pltpu.make_async_copy(table_hbm.at[idx_v.at[pl.ds(0, CHUNK)]],
                              bufs.at[b], gsems[b]).wait()

    def write_start(j, b):
        pltpu.async_copy(bufs.at[b],
                         out_hbm.at[pl.ds(base + j * CHUNK, CHUNK)], wsems[b])

    def write_wait(b):
        pltpu.make_async_copy(bufs.at[b], out_hbm.at[pl.ds(base, CHUNK)],
                              wsems[b]).wait()

    pltpu.async_copy(table_hbm.at[idx_v.at[pl.ds(FULL_CHUNKS * CHUNK,
                                                 BIG_TAIL)]], tail_buf, gt)

    for j in range(NBUF - 1):
        gather_start(j, j)

    def chunk_body(jj, k):
        b = k % NBUF
        bg = (k + NBUF - 1) % NBUF

        @pl.when(jj >= 1)
        def _():
            write_wait(bg)

        @pl.when(jj + NBUF - 1 < FULL_CHUNKS)
        def _():
            gather_start(jj + NBUF - 1, bg)

        gather_wait(b)
        write_start(jj, b)

    def outer(i, carry):
        jj0 = i * NBUF
        for k in range(NBUF):
            chunk_body(jj0 + k, k)
        return carry

    lax.fori_loop(0, FULL_CHUNKS // NBUF, outer, 0)

    write_wait((FULL_CHUNKS - 1) % NBUF)

    pltpu.make_async_copy(table_hbm.at[idx_v.at[pl.ds(0, BIG_TAIL)]],
                          tail_buf, gt).wait()
    tail_off = base + FULL_CHUNKS * CHUNK

    @pl.when(is_big)
    def _():
        pltpu.async_copy(tail_buf, out_hbm.at[pl.ds(tail_off, BIG_TAIL)], wt)
        pltpu.make_async_copy(tail_buf, out_hbm.at[pl.ds(0, BIG_TAIL)],
                              wt).wait()

    @pl.when(jnp.logical_not(is_big))
    def _():
        pltpu.async_copy(tail_buf.at[pl.ds(0, SMALL_TAIL)],
                         out_hbm.at[pl.ds(tail_off, SMALL_TAIL)], wt)
        pltpu.make_async_copy(tail_buf.at[pl.ds(0, SMALL_TAIL)],
                              out_hbm.at[pl.ds(0, SMALL_TAIL)], wt).wait()


def kernel(table, permutation, subject_id=0):
    return _gather_kernel(table, permutation.astype(jnp.int32))
